# baseline (speedup 1.0000x reference)
import functools

import numpy as np
import jax
import jax.numpy as jnp
from jax import lax
from jax.experimental import pallas as pl
from jax.experimental.pallas import tpu as pltpu
from jax.experimental.pallas import tpu_sc as plsc

_B, _N, _D = 4, 2048, 1024
_K, _T = 256, 128
_L = 16
_NW = 32
_ANGLE_WEIGHT = 0.5


def _build_indices():
    rng = np.random.RandomState(0)
    k = min(_K, _N * (_N - 1) // 2)
    idx_i = rng.randint(0, _N, size=k).astype(np.int64)
    idx_j = rng.randint(0, _N - 1, size=k).astype(np.int64)
    idx_j = idx_j + (idx_j >= idx_i).astype(np.int64)
    t = min(_T, k)
    idx_k = rng.randint(0, _N, size=t).astype(np.int64)
    collides = (idx_k == idx_i[:t]) | (idx_k == idx_j[:t])
    while collides.any():
        idx_k[collides] = rng.randint(0, _N, size=int(collides.sum()))
        collides = (idx_k == idx_i[:t]) | (idx_k == idx_j[:t])
    return idx_i.astype(np.int32), idx_j.astype(np.int32), idx_k.astype(np.int32)


def _build_task_tables():
    ii, jj, kk = _build_indices()
    comb = np.zeros((_NW, 2, 3 * _L), np.int32)
    pair = np.zeros((_NW, 2, 2 * _L), np.int32)
    tasks_c = [(b, t) for b in range(_B) for t in range(_T)]
    tasks_p = [(b, p) for b in range(_B) for p in range(_T, _K)]
    for w in range(_NW):
        for t in range(_L):
            b, tc = tasks_c[w * _L + t]
            b2, tp = tasks_p[w * _L + t]
            for h in range(2):
                comb[w, h, 3 * t + 0] = 2 * (b * _N + ii[tc]) + h
                comb[w, h, 3 * t + 1] = 2 * (b * _N + jj[tc]) + h
                comb[w, h, 3 * t + 2] = 2 * (b * _N + kk[tc]) + h
                pair[w, h, 2 * t + 0] = 2 * (b2 * _N + ii[tp]) + h
                pair[w, h, 2 * t + 1] = 2 * (b2 * _N + jj[tp]) + h
    return comb, pair


_COMB_IDX, _PAIR_IDX = _build_task_tables()


def _rsqrt16(x):
    i = plsc.bitcast(x, jnp.int32)
    i = jnp.full((_L,), 0x5F3759DF, jnp.int32) - lax.shift_right_logical(
        i, jnp.full((_L,), 1, jnp.int32))
    y = plsc.bitcast(i, jnp.float32)
    for _ in range(4):
        y = y * (1.5 - 0.5 * x * y * y)
    return y


def _smooth_l1(d):
    a = jnp.abs(d)
    return jnp.where(a < 1.0, 0.5 * d * d, a - 0.5)


@functools.cache
def _make_kernel():
    mesh = plsc.VectorSubcoreMesh(core_axis_name="c", subcore_axis_name="s")
    return functools.partial(
        pl.kernel,
        mesh=mesh,
        out_type=jax.ShapeDtypeStruct((_NW, 2, _L), jnp.float32),
        compiler_params=pltpu.CompilerParams(needs_layout_passes=False),
        scratch_types=[
            pltpu.VMEM((2, 3 * _L), jnp.int32),
            pltpu.VMEM((2, 2 * _L), jnp.int32),
            pltpu.VMEM((3 * _L, _D // 2), jnp.float32),
            pltpu.VMEM((3 * _L, _D // 2), jnp.float32),
            pltpu.VMEM((3 * _L, _D // 2), jnp.float32),
            pltpu.VMEM((3 * _L, _D // 2), jnp.float32),
            pltpu.VMEM((2, _L), jnp.float32),
            pltpu.SemaphoreType.DMA,
            pltpu.SemaphoreType.DMA,
            pltpu.SemaphoreType.DMA,
            pltpu.SemaphoreType.DMA,
        ],
    )(_rel_loss_body)


def _rel_loss_body(s_hbm, t_hbm, cidx_hbm, pidx_hbm, out_hbm,
                   cidx_v, pidx_v, sA, sB, tA, tB, acc,
                   semA_s, semA_t, semB_s, semB_t):
    wid = lax.axis_index("s") * 2 + lax.axis_index("c")
    zero = jnp.zeros((_L,), jnp.float32)
    lanes = lax.iota(jnp.int32, _L)
    _DH = _D // 2
    bufs = [(sA, tA, semA_s, semA_t), (sB, tB, semB_s, semB_t)]

    pltpu.sync_copy(cidx_hbm.at[wid], cidx_v)
    pltpu.sync_copy(pidx_hbm.at[wid], pidx_v)

    def cstart(h):
        bs, bt, ses, set_ = bufs[h]
        return (pltpu.async_copy(s_hbm.at[cidx_v.at[h]], bs, ses),
                pltpu.async_copy(t_hbm.at[cidx_v.at[h]], bt, set_))

    def pstart(h):
        bs, bt, ses, set_ = bufs[h]
        return (pltpu.async_copy(s_hbm.at[pidx_v.at[h]],
                                 bs.at[pl.ds(0, 2 * _L), :], ses),
                pltpu.async_copy(t_hbm.at[pidx_v.at[h]],
                                 bt.at[pl.ds(0, 2 * _L), :], set_))

    row_i = lanes * 3
    row_j = lanes * 3 + 1
    row_k = lanes * 3 + 2

    def cload(bs, bt, d):
        col = jnp.bitwise_and(jnp.full((_L,), d, jnp.int32) + lanes * 65,
                              _DH - 1)
        si = plsc.load_gather(bs, [row_i, col])
        sj = plsc.load_gather(bs, [row_j, col])
        sk = plsc.load_gather(bs, [row_k, col])
        ti = plsc.load_gather(bt, [row_i, col])
        tj = plsc.load_gather(bt, [row_j, col])
        tk = plsc.load_gather(bt, [row_k, col])
        return (si - sj, sk - sj, ti - tj, tk - tj)

    cpend = [cstart(0), cstart(1)]
    carry = (zero,) * 9
    for h in range(2):
        bs, bt, _, _ = bufs[h]
        ca, cb = cpend[h]
        ca.wait()
        cb.wait()

        def cp1(d, c, bs=bs, bt=bt):
            ss_ij, ss_kj, st_ij, st_kj, sdr, tdr, cst, m_s, m_t = c
            dij_s, dkj_s, dij_t, dkj_t = cload(bs, bt, d)
            return (ss_ij + dij_s * dij_s, ss_kj + dkj_s * dkj_s,
                    st_ij + dij_t * dij_t, st_kj + dkj_t * dkj_t,
                    sdr + dij_s * dkj_s, tdr + dij_t * dkj_t,
                    cst + dij_s * dij_t,
                    jnp.maximum(m_s, jnp.abs(dij_s)),
                    jnp.maximum(m_t, jnp.abs(dij_t)))

        carry = lax.fori_loop(0, _DH, cp1, carry, unroll=8)

    (ss_ij, ss_kj, st_ij, st_kj, sdr, tdr, cst, m_s, m_t) = carry
    rs_ij = _rsqrt16(jnp.maximum(ss_ij, 1e-24))
    rs_kj = _rsqrt16(jnp.maximum(ss_kj, 1e-24))
    rt_ij = _rsqrt16(jnp.maximum(st_ij, 1e-24))
    rt_kj = _rsqrt16(jnp.maximum(st_kj, 1e-24))

    acc[0, :] = 0.5 * (ss_ij * rs_ij * rs_ij + st_ij * rt_ij * rt_ij
                       - 2.0 * cst * rs_ij * rt_ij)
    acc[1, :] = _smooth_l1(sdr * rs_ij * rs_kj - tdr * rt_ij * rt_kj)

    @pl.when(jnp.logical_not(jnp.all(m_s * rs_ij + m_t * rt_ij < 1.0)))
    def _c_exact():
        acc[0, :] = zero
        for h in range(2):
            bs, bt, _, _ = bufs[h]

            def cp2(d, lossv, bs=bs, bt=bt):
                dij_s, _, dij_t, _ = cload(bs, bt, d)
                return lossv + _smooth_l1(dij_s * rs_ij - dij_t * rt_ij)

            acc[0, :] = acc[0, :] + lax.fori_loop(0, _DH, cp2, zero, unroll=8)

    ppend = [pstart(0), pstart(1)]
    prow_i = lanes * 2
    prow_j = lanes * 2 + 1

    def pload(bs, bt, d):
        col = jnp.bitwise_and(jnp.full((_L,), d, jnp.int32) + lanes * 65,
                              _DH - 1)
        si = plsc.load_gather(bs, [prow_i, col])
        sj = plsc.load_gather(bs, [prow_j, col])
        ti = plsc.load_gather(bt, [prow_i, col])
        tj = plsc.load_gather(bt, [prow_j, col])
        return (si - sj, ti - tj)

    pcarry = (zero,) * 5
    for h in range(2):
        bs, bt, _, _ = bufs[h]
        pa, pb = ppend[h]
        pa.wait()
        pb.wait()

        def pp1(d, c, bs=bs, bt=bt):
            ss, st, cst_, pm_s, pm_t = c
            ds_, dt_ = pload(bs, bt, d)
            return (ss + ds_ * ds_, st + dt_ * dt_, cst_ + ds_ * dt_,
                    jnp.maximum(pm_s, jnp.abs(ds_)),
                    jnp.maximum(pm_t, jnp.abs(dt_)))

        pcarry = lax.fori_loop(0, _DH, pp1, pcarry, unroll=8)

    ss, st, pcst, pm_s, pm_t = pcarry
    prs = _rsqrt16(jnp.maximum(ss, 1e-24))
    prt = _rsqrt16(jnp.maximum(st, 1e-24))
    plossv = 0.5 * (ss * prs * prs + st * prt * prt - 2.0 * pcst * prs * prt)
    acc[0, :] = acc[0, :] + plossv

    @pl.when(jnp.logical_not(jnp.all(pm_s * prs + pm_t * prt < 1.0)))
    def _p_exact():
        acc[0, :] = acc[0, :] - plossv
        for h in range(2):
            bs, bt, _, _ = bufs[h]

            def pp2(d, lossv, bs=bs, bt=bt):
                ds_, dt_ = pload(bs, bt, d)
                return lossv + _smooth_l1(ds_ * prs - dt_ * prt)

            acc[0, :] = acc[0, :] + lax.fori_loop(0, _DH, pp2, zero, unroll=8)
    pltpu.sync_copy(acc, out_hbm.at[wid])


def kernel(student_tokens, teacher_tokens):
    s = student_tokens.reshape(_B * _N * 2, _D // 2)
    t = teacher_tokens.reshape(_B * _N * 2, _D // 2)
    out = _make_kernel()(s, t, jnp.asarray(_COMB_IDX), jnp.asarray(_PAIR_IDX))
    dist = out[:, 0, :].sum() / (_B * _K * _D)
    ang = out[:, 1, :].sum() / (_B * _T)
    return dist + _ANGLE_WEIGHT * ang


# speedup vs baseline: 2.3328x; 2.3328x over previous
import functools

import numpy as np
import jax
import jax.numpy as jnp
from jax import lax
from jax.experimental import pallas as pl
from jax.experimental.pallas import tpu as pltpu
from jax.experimental.pallas import tpu_sc as plsc

_B, _N, _D = 4, 2048, 1024
_K, _T = 256, 128
_L = 16
_NW = 32
_ANGLE_WEIGHT = 0.5


def _build_indices():
    rng = np.random.RandomState(0)
    k = min(_K, _N * (_N - 1) // 2)
    idx_i = rng.randint(0, _N, size=k).astype(np.int64)
    idx_j = rng.randint(0, _N - 1, size=k).astype(np.int64)
    idx_j = idx_j + (idx_j >= idx_i).astype(np.int64)
    t = min(_T, k)
    idx_k = rng.randint(0, _N, size=t).astype(np.int64)
    collides = (idx_k == idx_i[:t]) | (idx_k == idx_j[:t])
    while collides.any():
        idx_k[collides] = rng.randint(0, _N, size=int(collides.sum()))
        collides = (idx_k == idx_i[:t]) | (idx_k == idx_j[:t])
    return idx_i.astype(np.int32), idx_j.astype(np.int32), idx_k.astype(np.int32)


def _build_task_tables():
    ii, jj, kk = _build_indices()
    comb = np.zeros((_NW, 3 * _L), np.int32)
    pair = np.zeros((_NW, 2 * _L), np.int32)
    tasks_c = [(b, t) for b in range(_B) for t in range(_T)]
    tasks_p = [(b, p) for b in range(_B) for p in range(_T, _K)]
    for w in range(_NW):
        for t in range(_L):
            b, tc = tasks_c[w * _L + t]
            comb[w, 3 * t + 0] = b * _N + ii[tc]
            comb[w, 3 * t + 1] = b * _N + jj[tc]
            comb[w, 3 * t + 2] = b * _N + kk[tc]
            b, tp = tasks_p[w * _L + t]
            pair[w, 2 * t + 0] = b * _N + ii[tp]
            pair[w, 2 * t + 1] = b * _N + jj[tp]
    return comb, pair


_COMB_IDX, _PAIR_IDX = _build_task_tables()


def _rsqrt16(x):
    i = plsc.bitcast(x, jnp.int32)
    i = jnp.full((_L,), 0x5F3759DF, jnp.int32) - lax.shift_right_logical(
        i, jnp.full((_L,), 1, jnp.int32))
    y = plsc.bitcast(i, jnp.float32)
    for _ in range(4):
        y = y * (1.5 - 0.5 * x * y * y)
    return y


def _smooth_l1(d):
    a = jnp.abs(d)
    return jnp.where(a < 1.0, 0.5 * d * d, a - 0.5)


@functools.cache
def _make_kernel():
    mesh = plsc.VectorSubcoreMesh(core_axis_name="c", subcore_axis_name="s")
    return functools.partial(
        pl.kernel,
        mesh=mesh,
        out_type=jax.ShapeDtypeStruct((_NW, 2, _L), jnp.float32),
        compiler_params=pltpu.CompilerParams(needs_layout_passes=False),
        scratch_types=[
            pltpu.VMEM((3 * _L,), jnp.int32),
            pltpu.VMEM((2 * _L,), jnp.int32),
            pltpu.VMEM((3 * _L, _D), jnp.float32),
            pltpu.VMEM((3 * _L, _D), jnp.float32),
            pltpu.VMEM((2, _L), jnp.float32),
            pltpu.SemaphoreType.DMA,
            pltpu.SemaphoreType.DMA,
        ],
    )(_rel_loss_body)


def _rel_loss_body(s_hbm, t_hbm, cidx_hbm, pidx_hbm, out_hbm,
                   cidx_v, pidx_v, rs_v, rt_v, acc, sem_s, sem_t):
    wid = lax.axis_index("s") * 2 + lax.axis_index("c")
    zero = jnp.zeros((_L,), jnp.float32)
    lanes = lax.iota(jnp.int32, _L)

    pltpu.sync_copy(cidx_hbm.at[wid], cidx_v)
    cp_s = pltpu.async_copy(s_hbm.at[cidx_v], rs_v, sem_s)
    cp_t = pltpu.async_copy(t_hbm.at[cidx_v], rt_v, sem_t)
    cp_s.wait()
    cp_t.wait()

    row_i = lanes * 3
    row_j = lanes * 3 + 1
    row_k = lanes * 3 + 2

    def cload(d):
        col = jnp.bitwise_and(jnp.full((_L,), d, jnp.int32) + lanes * 65, _D - 1)
        si = plsc.load_gather(rs_v, [row_i, col])
        sj = plsc.load_gather(rs_v, [row_j, col])
        sk = plsc.load_gather(rs_v, [row_k, col])
        ti = plsc.load_gather(rt_v, [row_i, col])
        tj = plsc.load_gather(rt_v, [row_j, col])
        tk = plsc.load_gather(rt_v, [row_k, col])
        return (si - sj, sk - sj, ti - tj, tk - tj)

    def cp1(d, carry):
        ss_ij, ss_kj, st_ij, st_kj, sdr, tdr, cst, m_s, m_t = carry
        dij_s, dkj_s, dij_t, dkj_t = cload(d)
        return (ss_ij + dij_s * dij_s, ss_kj + dkj_s * dkj_s,
                st_ij + dij_t * dij_t, st_kj + dkj_t * dkj_t,
                sdr + dij_s * dkj_s, tdr + dij_t * dkj_t,
                cst + dij_s * dij_t,
                jnp.maximum(m_s, jnp.abs(dij_s)),
                jnp.maximum(m_t, jnp.abs(dij_t)))

    (ss_ij, ss_kj, st_ij, st_kj, sdr, tdr, cst, m_s, m_t) = lax.fori_loop(
        0, _D, cp1, (zero,) * 9, unroll=16)
    rs_ij = _rsqrt16(jnp.maximum(ss_ij, 1e-24))
    rs_kj = _rsqrt16(jnp.maximum(ss_kj, 1e-24))
    rt_ij = _rsqrt16(jnp.maximum(st_ij, 1e-24))
    rt_kj = _rsqrt16(jnp.maximum(st_kj, 1e-24))

    acc[0, :] = 0.5 * (ss_ij * rs_ij * rs_ij + st_ij * rt_ij * rt_ij
                       - 2.0 * cst * rs_ij * rt_ij)
    acc[1, :] = _smooth_l1(sdr * rs_ij * rs_kj - tdr * rt_ij * rt_kj)

    @pl.when(jnp.logical_not(jnp.all(m_s * rs_ij + m_t * rt_ij < 1.0)))
    def _c_exact():
        def cp2(d, lossv):
            dij_s, _, dij_t, _ = cload(d)
            return lossv + _smooth_l1(dij_s * rs_ij - dij_t * rt_ij)

        acc[0, :] = lax.fori_loop(0, _D, cp2, zero, unroll=16)

    pltpu.sync_copy(pidx_hbm.at[wid], pidx_v)
    cp_s = pltpu.async_copy(s_hbm.at[pidx_v], rs_v.at[pl.ds(0, 2 * _L), :], sem_s)
    cp_t = pltpu.async_copy(t_hbm.at[pidx_v], rt_v.at[pl.ds(0, 2 * _L), :], sem_t)
    cp_s.wait()
    cp_t.wait()

    prow_i = lanes * 2
    prow_j = lanes * 2 + 1

    def pload(d):
        col = jnp.bitwise_and(jnp.full((_L,), d, jnp.int32) + lanes * 65, _D - 1)
        si = plsc.load_gather(rs_v, [prow_i, col])
        sj = plsc.load_gather(rs_v, [prow_j, col])
        ti = plsc.load_gather(rt_v, [prow_i, col])
        tj = plsc.load_gather(rt_v, [prow_j, col])
        return (si - sj, ti - tj)

    def pp1(d, carry):
        ss, st, cst, m_s, m_t = carry
        ds_, dt_ = pload(d)
        return (ss + ds_ * ds_, st + dt_ * dt_, cst + ds_ * dt_,
                jnp.maximum(m_s, jnp.abs(ds_)), jnp.maximum(m_t, jnp.abs(dt_)))

    ss, st, pcst, pm_s, pm_t = lax.fori_loop(0, _D, pp1, (zero,) * 5, unroll=16)
    prs = _rsqrt16(jnp.maximum(ss, 1e-24))
    prt = _rsqrt16(jnp.maximum(st, 1e-24))
    plossv = 0.5 * (ss * prs * prs + st * prt * prt - 2.0 * pcst * prs * prt)
    acc[0, :] = acc[0, :] + plossv

    @pl.when(jnp.logical_not(jnp.all(pm_s * prs + pm_t * prt < 1.0)))
    def _p_exact():
        def pp2(d, lossv):
            ds_, dt_ = pload(d)
            return lossv + _smooth_l1(ds_ * prs - dt_ * prt)

        acc[0, :] = acc[0, :] - plossv + lax.fori_loop(0, _D, pp2, zero, unroll=16)
    pltpu.sync_copy(acc, out_hbm.at[wid])


def kernel(student_tokens, teacher_tokens):
    s = student_tokens.reshape(_B * _N, _D)
    t = teacher_tokens.reshape(_B * _N, _D)
    out = _make_kernel()(s, t, jnp.asarray(_COMB_IDX), jnp.asarray(_PAIR_IDX))
    dist = out[:, 0, :].sum() / (_B * _K * _D)
    ang = out[:, 1, :].sum() / (_B * _T)
    return dist + _ANGLE_WEIGHT * ang


# speedup vs baseline: 2.4500x; 1.0502x over previous
import functools

import numpy as np
import jax
import jax.numpy as jnp
from jax import lax
from jax.experimental import pallas as pl
from jax.experimental.pallas import tpu as pltpu
from jax.experimental.pallas import tpu_sc as plsc

_B, _N, _D = 4, 2048, 1024
_K, _T = 256, 128
_L = 16
_NW = 32
_ANGLE_WEIGHT = 0.5


def _build_indices():
    rng = np.random.RandomState(0)
    k = min(_K, _N * (_N - 1) // 2)
    idx_i = rng.randint(0, _N, size=k).astype(np.int64)
    idx_j = rng.randint(0, _N - 1, size=k).astype(np.int64)
    idx_j = idx_j + (idx_j >= idx_i).astype(np.int64)
    t = min(_T, k)
    idx_k = rng.randint(0, _N, size=t).astype(np.int64)
    collides = (idx_k == idx_i[:t]) | (idx_k == idx_j[:t])
    while collides.any():
        idx_k[collides] = rng.randint(0, _N, size=int(collides.sum()))
        collides = (idx_k == idx_i[:t]) | (idx_k == idx_j[:t])
    return idx_i.astype(np.int32), idx_j.astype(np.int32), idx_k.astype(np.int32)


def _build_task_tables():
    ii, jj, kk = _build_indices()
    comb = np.zeros((_NW, 3 * _L), np.int32)
    pair = np.zeros((_NW, 2 * _L), np.int32)
    tasks_c = [(b, t) for b in range(_B) for t in range(_T)]
    tasks_p = [(b, p) for b in range(_B) for p in range(_T, _K)]
    for w in range(_NW):
        for t in range(_L):
            b, tc = tasks_c[w * _L + t]
            comb[w, 3 * t + 0] = b * _N + ii[tc]
            comb[w, 3 * t + 1] = b * _N + jj[tc]
            comb[w, 3 * t + 2] = b * _N + kk[tc]
            b, tp = tasks_p[w * _L + t]
            pair[w, 2 * t + 0] = b * _N + ii[tp]
            pair[w, 2 * t + 1] = b * _N + jj[tp]
    return comb, pair


_COMB_IDX, _PAIR_IDX = _build_task_tables()


def _rsqrt16(x):
    i = plsc.bitcast(x, jnp.int32)
    i = jnp.full((_L,), 0x5F3759DF, jnp.int32) - lax.shift_right_logical(
        i, jnp.full((_L,), 1, jnp.int32))
    y = plsc.bitcast(i, jnp.float32)
    for _ in range(4):
        y = y * (1.5 - 0.5 * x * y * y)
    return y


def _smooth_l1(d):
    a = jnp.abs(d)
    return jnp.where(a < 1.0, 0.5 * d * d, a - 0.5)


@functools.cache
def _make_kernel():
    mesh = plsc.VectorSubcoreMesh(core_axis_name="c", subcore_axis_name="s")
    return functools.partial(
        pl.kernel,
        mesh=mesh,
        out_type=jax.ShapeDtypeStruct((_NW, 2, _L), jnp.float32),
        compiler_params=pltpu.CompilerParams(needs_layout_passes=False),
        scratch_types=[
            pltpu.VMEM((3 * _L,), jnp.int32),
            pltpu.VMEM((2 * _L,), jnp.int32),
            pltpu.VMEM((3 * _L, _D), jnp.float32),
            pltpu.VMEM((3 * _L, _D), jnp.float32),
            pltpu.VMEM((2, _L), jnp.float32),
            pltpu.SemaphoreType.DMA,
            pltpu.SemaphoreType.DMA,
        ],
    )(_rel_loss_body)


def _rel_loss_body(s_hbm, t_hbm, cidx_hbm, pidx_hbm, out_hbm,
                   cidx_v, pidx_v, rs_v, rt_v, acc, sem_s, sem_t):
    wid = lax.axis_index("s") * 2 + lax.axis_index("c")
    zero = jnp.zeros((_L,), jnp.float32)
    lanes = lax.iota(jnp.int32, _L)

    pltpu.sync_copy(cidx_hbm.at[wid], cidx_v)
    cp_s = pltpu.async_copy(s_hbm.at[cidx_v], rs_v, sem_s)
    cp_t = pltpu.async_copy(t_hbm.at[cidx_v], rt_v, sem_t)
    cp_s.wait()
    cp_t.wait()

    row_i = lanes * 3
    row_j = lanes * 3 + 1
    row_k = lanes * 3 + 2

    def cload(d):
        col = jnp.bitwise_and(jnp.full((_L,), d, jnp.int32) + lanes * 65, _D - 1)
        si = plsc.load_gather(rs_v, [row_i, col])
        sj = plsc.load_gather(rs_v, [row_j, col])
        sk = plsc.load_gather(rs_v, [row_k, col])
        ti = plsc.load_gather(rt_v, [row_i, col])
        tj = plsc.load_gather(rt_v, [row_j, col])
        tk = plsc.load_gather(rt_v, [row_k, col])
        return (si - sj, sk - sj, ti - tj, tk - tj)

    def cp1(d, carry):
        ss_ij, ss_kj, st_ij, st_kj, sdr, tdr, cst, m_s, m_t = carry
        dij_s, dkj_s, dij_t, dkj_t = cload(d)
        return (ss_ij + dij_s * dij_s, ss_kj + dkj_s * dkj_s,
                st_ij + dij_t * dij_t, st_kj + dkj_t * dkj_t,
                sdr + dij_s * dkj_s, tdr + dij_t * dkj_t,
                cst + dij_s * dij_t,
                jnp.maximum(m_s, jnp.abs(dij_s)),
                jnp.maximum(m_t, jnp.abs(dij_t)))

    (ss_ij, ss_kj, st_ij, st_kj, sdr, tdr, cst, m_s, m_t) = lax.fori_loop(
        0, _D, cp1, (zero,) * 9, unroll=8)
    rs_ij = _rsqrt16(jnp.maximum(ss_ij, 1e-24))
    rs_kj = _rsqrt16(jnp.maximum(ss_kj, 1e-24))
    rt_ij = _rsqrt16(jnp.maximum(st_ij, 1e-24))
    rt_kj = _rsqrt16(jnp.maximum(st_kj, 1e-24))

    acc[0, :] = 0.5 * (ss_ij * rs_ij * rs_ij + st_ij * rt_ij * rt_ij
                       - 2.0 * cst * rs_ij * rt_ij)
    acc[1, :] = _smooth_l1(sdr * rs_ij * rs_kj - tdr * rt_ij * rt_kj)

    @pl.when(jnp.logical_not(jnp.all(m_s * rs_ij + m_t * rt_ij < 1.0)))
    def _c_exact():
        def cp2(d, lossv):
            dij_s, _, dij_t, _ = cload(d)
            return lossv + _smooth_l1(dij_s * rs_ij - dij_t * rt_ij)

        acc[0, :] = lax.fori_loop(0, _D, cp2, zero, unroll=8)

    pltpu.sync_copy(pidx_hbm.at[wid], pidx_v)
    cp_s = pltpu.async_copy(s_hbm.at[pidx_v], rs_v.at[pl.ds(0, 2 * _L), :], sem_s)
    cp_t = pltpu.async_copy(t_hbm.at[pidx_v], rt_v.at[pl.ds(0, 2 * _L), :], sem_t)
    cp_s.wait()
    cp_t.wait()

    prow_i = lanes * 2
    prow_j = lanes * 2 + 1

    def pload(d):
        col = jnp.bitwise_and(jnp.full((_L,), d, jnp.int32) + lanes * 65, _D - 1)
        si = plsc.load_gather(rs_v, [prow_i, col])
        sj = plsc.load_gather(rs_v, [prow_j, col])
        ti = plsc.load_gather(rt_v, [prow_i, col])
        tj = plsc.load_gather(rt_v, [prow_j, col])
        return (si - sj, ti - tj)

    def pp1(d, carry):
        ss, st, cst, m_s, m_t = carry
        ds_, dt_ = pload(d)
        return (ss + ds_ * ds_, st + dt_ * dt_, cst + ds_ * dt_,
                jnp.maximum(m_s, jnp.abs(ds_)), jnp.maximum(m_t, jnp.abs(dt_)))

    ss, st, pcst, pm_s, pm_t = lax.fori_loop(0, _D, pp1, (zero,) * 5, unroll=8)
    prs = _rsqrt16(jnp.maximum(ss, 1e-24))
    prt = _rsqrt16(jnp.maximum(st, 1e-24))
    plossv = 0.5 * (ss * prs * prs + st * prt * prt - 2.0 * pcst * prs * prt)
    acc[0, :] = acc[0, :] + plossv

    @pl.when(jnp.logical_not(jnp.all(pm_s * prs + pm_t * prt < 1.0)))
    def _p_exact():
        def pp2(d, lossv):
            ds_, dt_ = pload(d)
            return lossv + _smooth_l1(ds_ * prs - dt_ * prt)

        acc[0, :] = acc[0, :] - plossv + lax.fori_loop(0, _D, pp2, zero, unroll=8)
    pltpu.sync_copy(acc, out_hbm.at[wid])


def kernel(student_tokens, teacher_tokens):
    s = student_tokens.reshape(_B * _N, _D)
    t = teacher_tokens.reshape(_B * _N, _D)
    out = _make_kernel()(s, t, jnp.asarray(_COMB_IDX), jnp.asarray(_PAIR_IDX))
    dist = out[:, 0, :].sum() / (_B * _K * _D)
    ang = out[:, 1, :].sum() / (_B * _T)
    return dist + _ANGLE_WEIGHT * ang
